# Initial kernel scaffold; baseline (speedup 1.0000x reference)
import jax
import jax.numpy as jnp
from jax.experimental import pallas as pl


def kernel(target, contexts, negatives, in_embed, out_embed):
    raise NotImplementedError("write your pallas kernel here")


# speedup vs baseline: 3.4806x; 3.4806x over previous
import functools

import jax
import jax.numpy as jnp
from jax import lax
from jax.experimental import pallas as pl
from jax.experimental.pallas import tpu as pltpu
from jax.experimental.pallas import tpu_sc as plsc

_NC, _NS, _L = 2, 16, 16
_NW = _NC * _NS


def _softplus(x):
    e = jnp.exp(-jnp.abs(x))
    z = e / (e + 2.0)
    z2 = z * z
    p = jnp.float32(1.0 / 13.0)
    for c in (1.0 / 11.0, 1.0 / 9.0, 1.0 / 7.0, 1.0 / 5.0, 1.0 / 3.0, 1.0):
        p = p * z2 + jnp.float32(c)
    return jnp.maximum(x, 0.0) + 2.0 * z * p


def _build(B, C, K, D, interpret=False):
    per_w = B // _NW
    ng = per_w // _L
    rpg = _L * C // 64
    assert per_w % _L == 0 and ng % 2 == 0 and (_L * C) % 64 == 0 and C == K

    mesh = plsc.VectorSubcoreMesh(
        core_axis_name="c", subcore_axis_name="s",
        num_cores=_NC, num_subcores=_NS)

    scratch = [
        pltpu.VMEM((rpg * ng, 64), jnp.int32),
        pltpu.VMEM((rpg * ng, 64), jnp.int32),
        pltpu.VMEM((ng, _L), jnp.int32),
        pltpu.VMEM((_L * C, D), jnp.float32),
        pltpu.VMEM((_L * C, D), jnp.float32),
        pltpu.VMEM((_L * K, D), jnp.float32),
        pltpu.VMEM((_L * K, D), jnp.float32),
        pltpu.VMEM((_L, D), jnp.float32),
        pltpu.VMEM((_L, D), jnp.float32),
        pltpu.VMEM((_L,), jnp.float32),
        pltpu.SemaphoreType.DMA,
        pltpu.SemaphoreType.DMA,
    ]

    @functools.partial(
        pl.kernel,
        out_type=jax.ShapeDtypeStruct((_NW, _L), jnp.float32),
        mesh=mesh,
        scratch_types=scratch,
        compiler_params=pltpu.CompilerParams(
            needs_layout_passes=False, use_tc_tiling_on_sc=False),
        interpret=interpret,
    )
    def cbow_kernel(tgt_hbm, ctx_hbm, neg_hbm, in_e, out_e, out_hbm,
                    ctx_idx, neg_idx, tgt_idx, cr0, cr1, nr0, nr1,
                    tr0, tr1, loss_v, gsem0, gsem1):
        crows, nrows, trows = [cr0, cr1], [nr0, nr1], [tr0, tr1]
        gsem = [gsem0, gsem1]
        wid = lax.axis_index("s") * _NC + lax.axis_index("c")

        pltpu.sync_copy(ctx_hbm.at[pl.ds(wid * rpg * ng, rpg * ng)], ctx_idx)
        pltpu.sync_copy(neg_hbm.at[pl.ds(wid * rpg * ng, rpg * ng)], neg_idx)
        pltpu.sync_copy(tgt_hbm.at[pl.ds(wid * ng, ng)], tgt_idx)

        ii = lax.iota(jnp.int32, _L)
        rowc = [ii * C + c for c in range(C)]

        def issue(g, s):
            for j in range(rpg):
                pltpu.async_copy(in_e.at[ctx_idx.at[rpg * g + j]],
                                 crows[s].at[pl.ds(64 * j, 64)], gsem[s])
            for j in range(rpg):
                pltpu.async_copy(out_e.at[neg_idx.at[rpg * g + j]],
                                 nrows[s].at[pl.ds(64 * j, 64)], gsem[s])
            pltpu.async_copy(out_e.at[tgt_idx.at[g]], trows[s], gsem[s])

        def drain(g, s):
            for j in range(rpg):
                pltpu.make_async_copy(in_e.at[ctx_idx.at[rpg * g + j]],
                                      crows[s].at[pl.ds(64 * j, 64)],
                                      gsem[s]).wait()
            for j in range(rpg):
                pltpu.make_async_copy(out_e.at[neg_idx.at[rpg * g + j]],
                                      nrows[s].at[pl.ds(64 * j, 64)],
                                      gsem[s]).wait()
            pltpu.make_async_copy(out_e.at[tgt_idx.at[g]], trows[s],
                                  gsem[s]).wait()

        def compute(s, loss):
            cr, nr, tr = crows[s], nrows[s], trows[s]
            zeros = jnp.zeros((_L,), jnp.float32)

            def dbody(d, carry):
                accp, accn = carry
                col = jnp.broadcast_to(d, (_L,))
                vals = [plsc.load_gather(cr, [rowc[c], col]) for c in range(C)]
                while len(vals) > 1:
                    nxt = [vals[i] + vals[i + 1]
                           for i in range(0, len(vals) - 1, 2)]
                    if len(vals) % 2:
                        nxt.append(vals[-1])
                    vals = nxt
                hv = vals[0] * jnp.float32(1.0 / C)
                tv = plsc.load_gather(tr, [ii, col])
                accn = tuple(
                    accn[k] + hv * plsc.load_gather(nr, [rowc[k], col])
                    for k in range(K))
                return (accp + hv * tv, accn)

            accp, accn = lax.fori_loop(
                0, D, dbody, (zeros, tuple(zeros for _ in range(K))))
            total = _softplus(-accp)
            for k in range(K):
                total = total + _softplus(accn[k])
            return loss + total

        def gbody(i, loss):
            g0 = i * 2
            issue(g0 + 1, 1)
            drain(g0, 0)
            loss = compute(0, loss)
            issue((g0 + 2) % ng, 0)
            drain(g0 + 1, 1)
            loss = compute(1, loss)
            return loss

        issue(0, 0)
        loss = lax.fori_loop(0, ng // 2, gbody, jnp.zeros((_L,), jnp.float32))
        drain(0, 0)
        loss_v[...] = loss
        pltpu.sync_copy(loss_v, out_hbm.at[wid])

    return cbow_kernel


def kernel(target, contexts, negatives, in_embed, out_embed):
    B, C = contexts.shape
    K = negatives.shape[1]
    D = in_embed.shape[1]
    sc_call = _build(B, C, K, D)
    part = sc_call(
        target.reshape(B // _L, _L),
        contexts.reshape(B * C // 64, 64),
        negatives.reshape(B * K // 64, 64),
        in_embed,
        out_embed,
    )
    return jnp.sum(part) * jnp.float32(1.0 / B)


# speedup vs baseline: 5.7992x; 1.6662x over previous
import functools

import jax
import jax.numpy as jnp
from jax import lax
from jax.experimental import pallas as pl
from jax.experimental.pallas import tpu as pltpu
from jax.experimental.pallas import tpu_sc as plsc

_NC, _NS, _L = 2, 16, 16
_NW = _NC * _NS


def _softplus(x):
    e = jnp.exp(-jnp.abs(x))
    z = e / (e + 2.0)
    z2 = z * z
    p = jnp.float32(1.0 / 13.0)
    for c in (1.0 / 11.0, 1.0 / 9.0, 1.0 / 7.0, 1.0 / 5.0, 1.0 / 3.0, 1.0):
        p = p * z2 + jnp.float32(c)
    return jnp.maximum(x, 0.0) + 2.0 * z * p


def _build(B, C, K, D, interpret=False):
    per_w = B // _NW
    ng = per_w // _L
    assert per_w % _L == 0 and ng % 2 == 0 and C == K

    mesh = plsc.VectorSubcoreMesh(
        core_axis_name="c", subcore_axis_name="s",
        num_cores=_NC, num_subcores=_NS)

    cs = (C + 7) // 8 * 8
    scratch = [
        pltpu.VMEM((per_w, cs), jnp.int32),
        pltpu.VMEM((per_w, C), jnp.int32),
        pltpu.VMEM((per_w, K), jnp.int32),
        pltpu.VMEM((per_w,), jnp.int32),
        pltpu.VMEM((_L * C, D), jnp.float32),
        pltpu.VMEM((_L * C, D), jnp.float32),
        pltpu.VMEM((_L * K, D), jnp.float32),
        pltpu.VMEM((_L * K, D), jnp.float32),
        pltpu.VMEM((_L, D), jnp.float32),
        pltpu.VMEM((_L, D), jnp.float32),
        pltpu.VMEM((_L, D), jnp.float32),
        pltpu.VMEM((_L,), jnp.float32),
        pltpu.SemaphoreType.DMA,
        pltpu.SemaphoreType.DMA,
    ]

    @functools.partial(
        pl.kernel,
        out_type=jax.ShapeDtypeStruct((_NW, _L), jnp.float32),
        mesh=mesh,
        scratch_types=scratch,
        compiler_params=pltpu.CompilerParams(
            needs_layout_passes=False, use_tc_tiling_on_sc=False,
            disable_bounds_checks=True),
        interpret=interpret,
    )
    def cbow_kernel(tgt_hbm, ctx_hbm, neg_hbm, in_e, out_e, out_hbm,
                    idx_stage, ctx_idx, neg_idx, tgt_idx, cr0, cr1, nr0, nr1,
                    tr0, tr1, h_v, loss_v, gsem0, gsem1):
        crows, nrows, trows = [cr0, cr1], [nr0, nr1], [tr0, tr1]
        gsem = [gsem0, gsem1]
        wid = lax.axis_index("s") * _NC + lax.axis_index("c")

        ii = lax.iota(jnp.int32, _L)
        colhi = ii + (cs - _L)
        mhi = ii < (C - (cs - _L))

        def repack(dst, _unused):
            def rbody(r, _):
                lo = idx_stage[r, pl.ds(0, _L)]
                hi = idx_stage[r, pl.ds(cs - _L, _L)]
                dst[r, pl.ds(0, _L)] = lo + lo
                plsc.store_scatter(
                    dst, [jnp.broadcast_to(r, (_L,)), colhi], hi + hi,
                    mask=mhi)
                return 0
            lax.fori_loop(0, per_w, rbody, 0)

        pltpu.sync_copy(
            ctx_hbm.at[pl.ds(wid * per_w, per_w), pl.ds(0, cs)], idx_stage)
        repack(ctx_idx, None)
        pltpu.sync_copy(
            neg_hbm.at[pl.ds(wid * per_w, per_w), pl.ds(0, cs)], idx_stage)
        repack(neg_idx, None)
        pltpu.sync_copy(tgt_hbm.at[pl.ds(wid * per_w, per_w)], tgt_idx)

        def tdouble(r, _):
            t = tgt_idx[pl.ds(r * _L, _L)]
            tgt_idx[pl.ds(r * _L, _L)] = t + t
            return 0

        lax.fori_loop(0, ng, tdouble, 0)
        iiCD = ii * (C * D)
        zrow = jnp.zeros((_L,), jnp.int32)

        def issue(g, s):
            for j in range(_L):
                pltpu.async_copy(in_e.at[ctx_idx.at[g * _L + j]],
                                 crows[s].at[pl.ds(C * j, C)], gsem[s])
            for j in range(_L):
                pltpu.async_copy(out_e.at[neg_idx.at[g * _L + j]],
                                 nrows[s].at[pl.ds(K * j, K)], gsem[s])
            pltpu.async_copy(out_e.at[tgt_idx.at[pl.ds(g * _L, _L)]],
                             trows[s], gsem[s])

        def drain(g, s):
            for j in range(_L):
                pltpu.make_async_copy(in_e.at[ctx_idx.at[g * _L + j]],
                                      crows[s].at[pl.ds(C * j, C)],
                                      gsem[s]).wait()
            for j in range(_L):
                pltpu.make_async_copy(out_e.at[neg_idx.at[g * _L + j]],
                                      nrows[s].at[pl.ds(K * j, K)],
                                      gsem[s]).wait()
            pltpu.make_async_copy(out_e.at[tgt_idx.at[pl.ds(g * _L, _L)]],
                                  trows[s], gsem[s]).wait()

        def compute(s, loss):
            cr, nr, tr = crows[s], nrows[s], trows[s]
            zeros = jnp.zeros((_L,), jnp.float32)

            def pbody(b, _):
                row0 = b * C
                hj = [zeros] * (D // _L)
                for c in range(C):
                    for j in range(D // _L):
                        hj[j] = hj[j] + cr[row0 + c, pl.ds(j * _L, _L)]
                for j in range(D // _L):
                    h_v[b, pl.ds(j * _L, _L)] = hj[j] * jnp.float32(1.0 / C)
                return 0

            lax.fori_loop(0, _L, pbody, 0)

            def dbody(d, carry):
                accp, accn = carry
                cold = (d + ii) & (D - 1)
                flat0 = iiCD + cold
                hv = plsc.load_gather(h_v, [ii, cold])
                tv = plsc.load_gather(tr, [ii, cold])
                accn = tuple(
                    accn[k] + hv * plsc.load_gather(nr, [zrow, flat0 + k * D])
                    for k in range(K))
                return (accp + hv * tv, accn)

            accp, accn = lax.fori_loop(
                0, D, dbody, (zeros, tuple(zeros for _ in range(K))))
            total = _softplus(-accp)
            for k in range(K):
                total = total + _softplus(accn[k])
            return loss + total

        def gbody(i, loss):
            g0 = i * 2
            issue(g0 + 1, 1)
            drain(g0, 0)
            loss = compute(0, loss)
            issue((g0 + 2) % ng, 0)
            drain(g0 + 1, 1)
            loss = compute(1, loss)
            return loss

        issue(0, 0)
        loss = lax.fori_loop(0, ng // 2, gbody, jnp.zeros((_L,), jnp.float32))
        drain(0, 0)
        loss_v[...] = loss
        pltpu.sync_copy(loss_v, out_hbm.at[wid])

    return cbow_kernel


def kernel(target, contexts, negatives, in_embed, out_embed):
    B, C = contexts.shape
    K = negatives.shape[1]
    D = in_embed.shape[1]
    sc_call = _build(B, C, K, D)
    V = in_embed.shape[0]
    ctx128 = jnp.pad(contexts, ((0, 0), (0, 128 - C)))
    neg128 = jnp.pad(negatives, ((0, 0), (0, 128 - K)))
    in2 = jnp.pad(in_embed, ((0, 0), (0, 128 - D))).reshape(2 * V, D)
    out2 = jnp.pad(out_embed, ((0, 0), (0, 128 - D))).reshape(2 * V, D)
    part = sc_call(target, ctx128, neg128, in2, out2)
    return jnp.sum(part) * jnp.float32(1.0 / B)
